# baseline (speedup 1.0000x reference)
import functools

import jax
import jax.numpy as jnp
from jax.experimental import pallas as pl
from jax.experimental.pallas import tpu as pltpu

_B = 64
_V = 32768
_K = 8
_BG = 8
_G = _B // _BG

_K1 = (1832780943, 270669613)
_K2 = (64467757, 2916123636)

_ROT = (13, 15, 26, 6, 17, 29, 16, 24, 13, 15, 26, 6,
        17, 29, 16, 24, 13, 15, 26, 6)


def _threefry_bits(i, k0, k1):
    ks0 = jnp.uint32(k0)
    ks1 = jnp.uint32(k1)
    ks2 = jnp.uint32(k0 ^ k1 ^ 0x1BD11BDA)
    inject = ((ks1, ks2), (ks2, ks0), (ks0, ks1), (ks1, ks2), (ks2, ks0))
    x0 = jnp.full_like(i, ks0)
    x1 = i + ks1
    for r in range(20):
        rot = _ROT[r]
        x0 = x0 + x1
        x1 = (x1 << jnp.uint32(rot)) | (x1 >> jnp.uint32(32 - rot))
        x1 = x1 ^ x0
        if r % 4 == 3:
            a, b = inject[r // 4]
            x0 = x0 + a
            x1 = x1 + b + jnp.uint32(r // 4 + 1)
    return x0 ^ x1


def _tc_body(x_ref, z_ref, z2_ref, s1_ref, s2_ref):
    x = x_ref[...]
    x = jnp.where(jnp.isneginf(x), 0.0, x)
    l = jnp.sum(x, axis=1)
    z = jnp.where(l == 0.0, -1000.0, l)
    z_ref[...] = z
    z2_ref[...] = z

    g = pl.program_id(0)
    row = jax.lax.broadcasted_iota(jnp.uint32, (_BG, _V), 0)
    col = jax.lax.broadcasted_iota(jnp.uint32, (_BG, _V), 1)
    cnt = (jnp.uint32(_BG * _V) * jnp.uint32(g)
           + row * jnp.uint32(_V) + col)
    lin = jax.lax.broadcasted_iota(jnp.int32, (_BG, _V), 1)

    def samp(k0, k1):
        bits = _threefry_bits(cnt, k0, k1)
        fl = jax.lax.bitcast_convert_type(
            (bits >> jnp.uint32(9)) | jnp.uint32(0x3F800000), jnp.float32)
        fl = fl - jnp.float32(1.0)
        u = jnp.maximum(jnp.float32(1e-20),
                        fl * jnp.float32(1.0) + jnp.float32(1e-20))
        gn = -jnp.log(-jnp.log(u))
        n = z + gn
        mx = jnp.max(n, axis=1, keepdims=True)
        idx = jnp.min(jnp.where(n == mx, lin, jnp.int32(2**30)),
                      axis=1, keepdims=True)
        return jnp.broadcast_to(idx, (_BG, 128)).astype(jnp.int32)

    s1_ref[0] = samp(*_K1)
    s2_ref[0] = samp(*_K2)


@functools.partial(jax.jit)
def _run(logits):
    xt = jnp.swapaxes(logits, 1, 2)
    z, z2, s1, s2 = pl.pallas_call(
        _tc_body,
        grid=(_G,),
        in_specs=[
            pl.BlockSpec((_BG, _K, _V), lambda g: (g, 0, 0)),
        ],
        out_specs=[
            pl.BlockSpec((_BG, _V), lambda g: (g, 0)),
            pl.BlockSpec((_BG, _V), lambda g: (g, 0)),
            pl.BlockSpec((1, _BG, 128), lambda g: (g, 0, 0)),
            pl.BlockSpec((1, _BG, 128), lambda g: (g, 0, 0)),
        ],
        out_shape=[
            jax.ShapeDtypeStruct((_B, _V), jnp.float32),
            jax.ShapeDtypeStruct((_B, _V), jnp.float32),
            jax.ShapeDtypeStruct((_G, _BG, 128), jnp.int32),
            jax.ShapeDtypeStruct((_G, _BG, 128), jnp.int32),
        ],
        compiler_params=pltpu.CompilerParams(
            dimension_semantics=("parallel",),
        ),
    )(xt)
    return (z, z2, s1[:, :, 0].reshape(_B), s2[:, :, 0].reshape(_B))


def kernel(logits):
    return _run(logits)


# speedup vs baseline: 1.2296x; 1.2296x over previous
import functools

import jax
import jax.numpy as jnp
from jax.experimental import pallas as pl
from jax.experimental.pallas import tpu as pltpu

_B = 64
_V = 32768
_K = 8
_BG = 8
_G = _B // _BG
_CW = 512
_NC = _V // _CW

_K1 = (1832780943, 270669613)
_K2 = (64467757, 2916123636)

_ROT = (13, 15, 26, 6, 17, 29, 16, 24, 13, 15, 26, 6,
        17, 29, 16, 24, 13, 15, 26, 6)


def _threefry_bits(i, k0, k1):
    ks0 = jnp.uint32(k0)
    ks1 = jnp.uint32(k1)
    ks2 = jnp.uint32(k0 ^ k1 ^ 0x1BD11BDA)
    inject = ((ks1, ks2), (ks2, ks0), (ks0, ks1), (ks1, ks2), (ks2, ks0))
    x0 = jnp.full_like(i, ks0)
    x1 = i + ks1
    for r in range(20):
        rot = _ROT[r]
        x0 = x0 + x1
        x1 = (x1 << jnp.uint32(rot)) | (x1 >> jnp.uint32(32 - rot))
        x1 = x1 ^ x0
        if r % 4 == 3:
            a, b = inject[r // 4]
            x0 = x0 + a
            x1 = x1 + b + jnp.uint32(r // 4 + 1)
    return x0 ^ x1


def _gumbel(cnt, k0, k1):
    bits = _threefry_bits(cnt, k0, k1)
    fl = jax.lax.bitcast_convert_type(
        (bits >> jnp.uint32(9)) | jnp.uint32(0x3F800000), jnp.float32)
    fl = fl - jnp.float32(1.0)
    u = jnp.maximum(jnp.float32(1e-20),
                    fl * jnp.float32(1.0) + jnp.float32(1e-20))
    return -jnp.log(-jnp.log(u))


def _tc_body(x_ref, z_ref, z2_ref, s1_ref, s2_ref):
    g = pl.program_id(0)
    row = jax.lax.broadcasted_iota(jnp.uint32, (_BG, _CW), 0)
    col = jax.lax.broadcasted_iota(jnp.uint32, (_BG, _CW), 1)
    cnt0 = jnp.uint32(_BG * _V) * jnp.uint32(g) + row * jnp.uint32(_V) + col
    lin0 = jax.lax.broadcasted_iota(jnp.int32, (_BG, _CW), 1)
    neg = jnp.float32(-3.4e38)

    def chunk(c, carry):
        m1, i1, m2, i2 = carry
        sl = pl.ds(c * _CW, _CW)
        x = x_ref[:, :, sl]
        x = jnp.where(jnp.isneginf(x), 0.0, x)
        l = jnp.sum(x, axis=1)
        z = jnp.where(l == 0.0, -1000.0, l)
        z_ref[:, sl] = z
        z2_ref[:, sl] = z

        off = jnp.uint32(c * _CW)
        cnt = cnt0 + off
        lin = lin0 + c * _CW

        n1 = z + _gumbel(cnt, *_K1)
        up1 = n1 > m1
        m1 = jnp.where(up1, n1, m1)
        i1 = jnp.where(up1, lin, i1)

        n2 = z + _gumbel(cnt, *_K2)
        up2 = n2 > m2
        m2 = jnp.where(up2, n2, m2)
        i2 = jnp.where(up2, lin, i2)
        return (m1, i1, m2, i2)

    init = (jnp.full((_BG, _CW), neg, jnp.float32),
            jnp.zeros((_BG, _CW), jnp.int32),
            jnp.full((_BG, _CW), neg, jnp.float32),
            jnp.zeros((_BG, _CW), jnp.int32))
    m1, i1, m2, i2 = jax.lax.fori_loop(0, _NC, chunk, init)

    def finalize(m, i):
        gmax = jnp.max(m, axis=1, keepdims=True)
        idx = jnp.min(jnp.where(m == gmax, i, jnp.int32(2**30)),
                      axis=1, keepdims=True)
        return jnp.broadcast_to(idx, (_BG, 128)).astype(jnp.int32)

    s1_ref[0] = finalize(m1, i1)
    s2_ref[0] = finalize(m2, i2)


@functools.partial(jax.jit)
def _run(logits):
    xt = jnp.swapaxes(logits, 1, 2)
    z, z2, s1, s2 = pl.pallas_call(
        _tc_body,
        grid=(_G,),
        in_specs=[
            pl.BlockSpec((_BG, _K, _V), lambda g: (g, 0, 0)),
        ],
        out_specs=[
            pl.BlockSpec((_BG, _V), lambda g: (g, 0)),
            pl.BlockSpec((_BG, _V), lambda g: (g, 0)),
            pl.BlockSpec((1, _BG, 128), lambda g: (g, 0, 0)),
            pl.BlockSpec((1, _BG, 128), lambda g: (g, 0, 0)),
        ],
        out_shape=[
            jax.ShapeDtypeStruct((_B, _V), jnp.float32),
            jax.ShapeDtypeStruct((_B, _V), jnp.float32),
            jax.ShapeDtypeStruct((_G, _BG, 128), jnp.int32),
            jax.ShapeDtypeStruct((_G, _BG, 128), jnp.int32),
        ],
        compiler_params=pltpu.CompilerParams(
            dimension_semantics=("parallel",),
        ),
    )(xt)
    return (z, z2, s1[:, :, 0].reshape(_B), s2[:, :, 0].reshape(_B))


def kernel(logits):
    return _run(logits)


# speedup vs baseline: 1.2431x; 1.0110x over previous
import functools

import jax
import jax.numpy as jnp
from jax import lax
from jax.experimental import pallas as pl
from jax.experimental.pallas import tpu as pltpu
from jax.experimental.pallas import tpu_sc as plsc

_B = 64
_V = 32768
_K = 8
_BG = 8
_CW = 512
_NC = _V // _CW

_SC_ROWS = 16
_R0 = _B - _SC_ROWS
_GA = _R0 // _BG
_GB = _SC_ROWS // _BG
_HW = _V // 2

_K1 = (1832780943, 270669613)
_K2 = (64467757, 2916123636)

_ROT = (13, 15, 26, 6, 17, 29, 16, 24, 13, 15, 26, 6,
        17, 29, 16, 24, 13, 15, 26, 6)


def _threefry_bits(i, k0, k1):
    ks0 = jnp.uint32(k0)
    ks1 = jnp.uint32(k1)
    ks2 = jnp.uint32(k0 ^ k1 ^ 0x1BD11BDA)
    inject = ((ks1, ks2), (ks2, ks0), (ks0, ks1), (ks1, ks2), (ks2, ks0))
    x0 = jnp.full_like(i, ks0)
    x1 = i + ks1
    for r in range(20):
        rot = _ROT[r]
        x0 = x0 + x1
        x1 = (x1 << jnp.uint32(rot)) | (x1 >> jnp.uint32(32 - rot))
        x1 = x1 ^ x0
        if r % 4 == 3:
            a, b = inject[r // 4]
            x0 = x0 + a
            x1 = x1 + b + jnp.uint32(r // 4 + 1)
    return x0 ^ x1


def _bits_to_gumbel(bits):
    fl = jax.lax.bitcast_convert_type(
        (bits >> jnp.uint32(9)) | jnp.uint32(0x3F800000), jnp.float32)
    fl = fl - jnp.float32(1.0)
    u = jnp.maximum(jnp.float32(1e-20),
                    fl * jnp.float32(1.0) + jnp.float32(1e-20))
    return -jnp.log(-jnp.log(u))


def _sc_bits_body(o1, o2, buf1, buf2):
    cid = lax.axis_index("c")
    sid = lax.axis_index("s")
    wid = sid * 2 + cid
    row = wid // 2
    half = wid % 2
    base = (jnp.uint32(_R0 + row) * jnp.uint32(_V)
            + jnp.uint32(half) * jnp.uint32(_HW))
    iota = lax.iota(jnp.uint32, 16)

    def fill(buf, k0, k1):
        def body(j, _):
            b0 = base + (j * jnp.int32(64)).astype(jnp.uint32)
            for t in range(4):
                cvec = b0 + jnp.uint32(t * 16) + iota
                buf[pl.ds(j * 64 + t * 16, 16)] = _threefry_bits(cvec, k0, k1)
            return 0
        lax.fori_loop(0, _HW // 64, body, 0)

    fill(buf1, *_K1)
    pltpu.sync_copy(buf1, o1.at[row, pl.ds(half * _HW, _HW)])
    fill(buf2, *_K2)
    pltpu.sync_copy(buf2, o2.at[row, pl.ds(half * _HW, _HW)])


_sc_bits = functools.partial(
    pl.kernel,
    out_type=[jax.ShapeDtypeStruct((_SC_ROWS, _V), jnp.uint32),
              jax.ShapeDtypeStruct((_SC_ROWS, _V), jnp.uint32)],
    scratch_types=[pltpu.VMEM((_HW,), jnp.uint32),
                   pltpu.VMEM((_HW,), jnp.uint32)],
    mesh=plsc.VectorSubcoreMesh(core_axis_name="c", subcore_axis_name="s"),
)(_sc_bits_body)


def _fused_rows(x_ref, z_ref, z2_ref, s1_ref, s2_ref, bits1=None, bits2=None,
                row_base=0):
    g = pl.program_id(0)
    row = jax.lax.broadcasted_iota(jnp.uint32, (_BG, _CW), 0)
    col = jax.lax.broadcasted_iota(jnp.uint32, (_BG, _CW), 1)
    cnt0 = (jnp.uint32(_BG * _V) * jnp.uint32(g) + jnp.uint32(row_base * _V)
            + row * jnp.uint32(_V) + col)
    lin0 = jax.lax.broadcasted_iota(jnp.int32, (_BG, _CW), 1)
    neg = jnp.float32(-3.4e38)

    def chunk(c, carry):
        m1, i1, m2, i2 = carry
        sl = pl.ds(c * _CW, _CW)
        x = x_ref[:, :, sl]
        x = jnp.where(jnp.isneginf(x), 0.0, x)
        l = jnp.sum(x, axis=1)
        z = jnp.where(l == 0.0, -1000.0, l)
        z_ref[:, sl] = z
        z2_ref[:, sl] = z

        if bits1 is None:
            cnt = cnt0 + jnp.uint32(c * _CW)
            b1 = _threefry_bits(cnt, *_K1)
            b2 = _threefry_bits(cnt, *_K2)
        else:
            b1 = bits1[:, sl]
            b2 = bits2[:, sl]
        lin = lin0 + c * _CW

        n1 = z + _bits_to_gumbel(b1)
        up1 = n1 > m1
        m1 = jnp.where(up1, n1, m1)
        i1 = jnp.where(up1, lin, i1)

        n2 = z + _bits_to_gumbel(b2)
        up2 = n2 > m2
        m2 = jnp.where(up2, n2, m2)
        i2 = jnp.where(up2, lin, i2)
        return (m1, i1, m2, i2)

    init = (jnp.full((_BG, _CW), neg, jnp.float32),
            jnp.zeros((_BG, _CW), jnp.int32),
            jnp.full((_BG, _CW), neg, jnp.float32),
            jnp.zeros((_BG, _CW), jnp.int32))
    m1, i1, m2, i2 = jax.lax.fori_loop(0, _NC, chunk, init)

    def finalize(m, i):
        gmax = jnp.max(m, axis=1, keepdims=True)
        idx = jnp.min(jnp.where(m == gmax, i, jnp.int32(2**30)),
                      axis=1, keepdims=True)
        return jnp.broadcast_to(idx, (_BG, 128)).astype(jnp.int32)

    s1_ref[0] = finalize(m1, i1)
    s2_ref[0] = finalize(m2, i2)


def _tc_a_body(x_ref, z_ref, z2_ref, s1_ref, s2_ref):
    _fused_rows(x_ref, z_ref, z2_ref, s1_ref, s2_ref)


def _tc_b_body(x_ref, b1_ref, b2_ref, za_ref, z2a_ref,
               z_ref, z2_ref, s1_ref, s2_ref):
    _fused_rows(x_ref, z_ref, z2_ref, s1_ref, s2_ref,
                bits1=b1_ref, bits2=b2_ref, row_base=_R0)


@functools.partial(jax.jit)
def _run(logits):
    xt = jnp.swapaxes(logits, 1, 2)
    bits1, bits2 = _sc_bits()

    z, z2, s1h, s2h = pl.pallas_call(
        _tc_a_body,
        grid=(_GA,),
        in_specs=[pl.BlockSpec((_BG, _K, _V), lambda g: (g, 0, 0))],
        out_specs=[
            pl.BlockSpec((_BG, _V), lambda g: (g, 0)),
            pl.BlockSpec((_BG, _V), lambda g: (g, 0)),
            pl.BlockSpec((1, _BG, 128), lambda g: (g, 0, 0)),
            pl.BlockSpec((1, _BG, 128), lambda g: (g, 0, 0)),
        ],
        out_shape=[
            jax.ShapeDtypeStruct((_B, _V), jnp.float32),
            jax.ShapeDtypeStruct((_B, _V), jnp.float32),
            jax.ShapeDtypeStruct((_GA, _BG, 128), jnp.int32),
            jax.ShapeDtypeStruct((_GA, _BG, 128), jnp.int32),
        ],
        compiler_params=pltpu.CompilerParams(
            dimension_semantics=("parallel",),
        ),
    )(xt)

    zf, z2f, s1t, s2t = pl.pallas_call(
        _tc_b_body,
        grid=(_GB,),
        in_specs=[
            pl.BlockSpec((_BG, _K, _V), lambda g: (g + _GA, 0, 0)),
            pl.BlockSpec((_BG, _V), lambda g: (g, 0)),
            pl.BlockSpec((_BG, _V), lambda g: (g, 0)),
            pl.BlockSpec(memory_space=pl.ANY),
            pl.BlockSpec(memory_space=pl.ANY),
        ],
        out_specs=[
            pl.BlockSpec((_BG, _V), lambda g: (g + _GA, 0)),
            pl.BlockSpec((_BG, _V), lambda g: (g + _GA, 0)),
            pl.BlockSpec((1, _BG, 128), lambda g: (g, 0, 0)),
            pl.BlockSpec((1, _BG, 128), lambda g: (g, 0, 0)),
        ],
        out_shape=[
            jax.ShapeDtypeStruct((_B, _V), jnp.float32),
            jax.ShapeDtypeStruct((_B, _V), jnp.float32),
            jax.ShapeDtypeStruct((_GB, _BG, 128), jnp.int32),
            jax.ShapeDtypeStruct((_GB, _BG, 128), jnp.int32),
        ],
        input_output_aliases={3: 0, 4: 1},
        compiler_params=pltpu.CompilerParams(
            dimension_semantics=("parallel",),
        ),
    )(xt, bits1, bits2, z, z2)

    s1 = jnp.concatenate([s1h, s1t])[:, :, 0].reshape(_B)
    s2 = jnp.concatenate([s2h, s2t])[:, :, 0].reshape(_B)
    return (zf, z2f, s1, s2)


def kernel(logits):
    return _run(logits)


# speedup vs baseline: 1.4610x; 1.1753x over previous
import functools

import jax
import jax.numpy as jnp
from jax import lax
from jax.experimental import pallas as pl
from jax.experimental.pallas import tpu as pltpu
from jax.experimental.pallas import tpu_sc as plsc

_B = 64
_V = 32768
_K = 8
_BG = 8
_CW = 512
_NC = _V // _CW

_SC_ROWS = 16
_R0 = _B - _SC_ROWS
_GA = _R0 // _BG
_GB = _SC_ROWS // _BG
_HW = _V // 2

_K1 = (1832780943, 270669613)
_K2 = (64467757, 2916123636)

_ROT = (13, 15, 26, 6, 17, 29, 16, 24, 13, 15, 26, 6,
        17, 29, 16, 24, 13, 15, 26, 6)


def _threefry_bits(i, k0, k1):
    ks0 = jnp.uint32(k0)
    ks1 = jnp.uint32(k1)
    ks2 = jnp.uint32(k0 ^ k1 ^ 0x1BD11BDA)
    inject = ((ks1, ks2), (ks2, ks0), (ks0, ks1), (ks1, ks2), (ks2, ks0))
    x0 = jnp.full_like(i, ks0)
    x1 = i + ks1
    for r in range(20):
        rot = _ROT[r]
        x0 = x0 + x1
        x1 = (x1 << jnp.uint32(rot)) | (x1 >> jnp.uint32(32 - rot))
        x1 = x1 ^ x0
        if r % 4 == 3:
            a, b = inject[r // 4]
            x0 = x0 + a
            x1 = x1 + b + jnp.uint32(r // 4 + 1)
    return x0 ^ x1


def _bits_to_gumbel(bits):
    fl = jax.lax.bitcast_convert_type(
        (bits >> jnp.uint32(9)) | jnp.uint32(0x3F800000), jnp.float32)
    fl = fl - jnp.float32(1.0)
    u = jnp.maximum(jnp.float32(1e-20),
                    fl * jnp.float32(1.0) + jnp.float32(1e-20))
    return -jnp.log(-jnp.log(u))


def _sum_mask(x_ref, sl):
    l = None
    for k in range(_K):
        xk = x_ref[:, k, sl]
        xk = jnp.where(jnp.isneginf(xk), 0.0, xk)
        l = xk if l is None else l + xk
    return jnp.where(l == 0.0, -1000.0, l)


def _finalize(m, i):
    gmax = jnp.max(m, axis=1, keepdims=True)
    idx = jnp.min(jnp.where(m == gmax, i, jnp.int32(2**30)),
                  axis=1, keepdims=True)
    return jnp.broadcast_to(idx, (_BG, 128)).astype(jnp.int32)


_NEG = -3.4e38


def _sc_bits_body(o1, o2, buf1, buf2):
    cid = lax.axis_index("c")
    sid = lax.axis_index("s")
    wid = sid * 2 + cid
    row = wid // 2
    half = wid % 2
    base = (jnp.uint32(_R0 + row) * jnp.uint32(_V)
            + jnp.uint32(half) * jnp.uint32(_HW))
    iota = lax.iota(jnp.uint32, 16)

    def fill(buf, k0, k1):
        def body(j, _):
            b0 = base + (j * jnp.int32(64)).astype(jnp.uint32)
            for t in range(4):
                cvec = b0 + jnp.uint32(t * 16) + iota
                buf[pl.ds(j * 64 + t * 16, 16)] = _threefry_bits(cvec, k0, k1)
            return 0
        lax.fori_loop(0, _HW // 64, body, 0)

    fill(buf1, *_K1)
    pltpu.sync_copy(buf1, o1.at[row, pl.ds(half * _HW, _HW)])
    fill(buf2, *_K2)
    pltpu.sync_copy(buf2, o2.at[row, pl.ds(half * _HW, _HW)])


_sc_bits = functools.partial(
    pl.kernel,
    out_type=[jax.ShapeDtypeStruct((_SC_ROWS, _V), jnp.uint32),
              jax.ShapeDtypeStruct((_SC_ROWS, _V), jnp.uint32)],
    scratch_types=[pltpu.VMEM((_HW,), jnp.uint32),
                   pltpu.VMEM((_HW,), jnp.uint32)],
    mesh=plsc.VectorSubcoreMesh(core_axis_name="c", subcore_axis_name="s"),
)(_sc_bits_body)


def _tc_a_body(x_ref, z_ref, z2_ref, s1_ref, s2_ref):
    g = pl.program_id(0)
    lin0 = jax.lax.broadcasted_iota(jnp.int32, (_BG, _CW), 1)

    @pl.when(g < _GA)
    def _head():
        row = jax.lax.broadcasted_iota(jnp.uint32, (_BG, _CW), 0)
        col = jax.lax.broadcasted_iota(jnp.uint32, (_BG, _CW), 1)
        cnt0 = (jnp.uint32(_BG * _V) * jnp.uint32(g)
                + row * jnp.uint32(_V) + col)

        def loop1(c, carry):
            m1, i1 = carry
            sl = pl.ds(c * _CW, _CW)
            z = _sum_mask(x_ref, sl)
            z_ref[:, sl] = z
            z2_ref[:, sl] = z
            b1 = _threefry_bits(cnt0 + jnp.uint32(c * _CW), *_K1)
            n1 = z + _bits_to_gumbel(b1)
            lin = lin0 + c * _CW
            up = n1 > m1
            return (jnp.where(up, n1, m1), jnp.where(up, lin, i1))

        def loop2(c, carry):
            m2, i2 = carry
            sl = pl.ds(c * _CW, _CW)
            z = z_ref[:, sl]
            b2 = _threefry_bits(cnt0 + jnp.uint32(c * _CW), *_K2)
            n2 = z + _bits_to_gumbel(b2)
            lin = lin0 + c * _CW
            up = n2 > m2
            return (jnp.where(up, n2, m2), jnp.where(up, lin, i2))

        init = (jnp.full((_BG, _CW), _NEG, jnp.float32),
                jnp.zeros((_BG, _CW), jnp.int32))
        m1, i1 = jax.lax.fori_loop(0, _NC, loop1, init)
        m2, i2 = jax.lax.fori_loop(0, _NC, loop2, init)
        s1_ref[0] = _finalize(m1, i1)
        s2_ref[0] = _finalize(m2, i2)

    @pl.when(g >= _GA)
    def _tail():
        def loopz(c, carry):
            sl = pl.ds(c * _CW, _CW)
            z = _sum_mask(x_ref, sl)
            z_ref[:, sl] = z
            z2_ref[:, sl] = z
            return carry
        jax.lax.fori_loop(0, _NC, loopz, 0)


def _tc_b_body(z_ref, b1_ref, b2_ref, s1_ref, s2_ref):
    lin0 = jax.lax.broadcasted_iota(jnp.int32, (_BG, _CW), 1)

    def loop(c, carry):
        m1, i1, m2, i2 = carry
        sl = pl.ds(c * _CW, _CW)
        z = z_ref[:, sl]
        lin = lin0 + c * _CW
        n1 = z + _bits_to_gumbel(b1_ref[:, sl])
        up1 = n1 > m1
        m1 = jnp.where(up1, n1, m1)
        i1 = jnp.where(up1, lin, i1)
        n2 = z + _bits_to_gumbel(b2_ref[:, sl])
        up2 = n2 > m2
        m2 = jnp.where(up2, n2, m2)
        i2 = jnp.where(up2, lin, i2)
        return (m1, i1, m2, i2)

    init = (jnp.full((_BG, _CW), _NEG, jnp.float32),
            jnp.zeros((_BG, _CW), jnp.int32),
            jnp.full((_BG, _CW), _NEG, jnp.float32),
            jnp.zeros((_BG, _CW), jnp.int32))
    m1, i1, m2, i2 = jax.lax.fori_loop(0, _NC, loop, init)
    s1_ref[0] = _finalize(m1, i1)
    s2_ref[0] = _finalize(m2, i2)


@functools.partial(jax.jit)
def _run(logits):
    xt = jnp.swapaxes(logits, 1, 2)
    bits1, bits2 = _sc_bits()

    z, z2, s1h, s2h = pl.pallas_call(
        _tc_a_body,
        grid=(_B // _BG,),
        in_specs=[pl.BlockSpec((_BG, _K, _V), lambda g: (g, 0, 0))],
        out_specs=[
            pl.BlockSpec((_BG, _V), lambda g: (g, 0)),
            pl.BlockSpec((_BG, _V), lambda g: (g, 0)),
            pl.BlockSpec((1, _BG, 128), lambda g: (g, 0, 0)),
            pl.BlockSpec((1, _BG, 128), lambda g: (g, 0, 0)),
        ],
        out_shape=[
            jax.ShapeDtypeStruct((_B, _V), jnp.float32),
            jax.ShapeDtypeStruct((_B, _V), jnp.float32),
            jax.ShapeDtypeStruct((_B // _BG, _BG, 128), jnp.int32),
            jax.ShapeDtypeStruct((_B // _BG, _BG, 128), jnp.int32),
        ],
        compiler_params=pltpu.CompilerParams(
            dimension_semantics=("arbitrary",),
        ),
    )(xt)

    s1t, s2t = pl.pallas_call(
        _tc_b_body,
        grid=(_GB,),
        in_specs=[
            pl.BlockSpec((_BG, _V), lambda g: (g + _GA, 0)),
            pl.BlockSpec((_BG, _V), lambda g: (g, 0)),
            pl.BlockSpec((_BG, _V), lambda g: (g, 0)),
        ],
        out_specs=[
            pl.BlockSpec((1, _BG, 128), lambda g: (g, 0, 0)),
            pl.BlockSpec((1, _BG, 128), lambda g: (g, 0, 0)),
        ],
        out_shape=[
            jax.ShapeDtypeStruct((_GB, _BG, 128), jnp.int32),
            jax.ShapeDtypeStruct((_GB, _BG, 128), jnp.int32),
        ],
        compiler_params=pltpu.CompilerParams(
            dimension_semantics=("arbitrary",),
        ),
    )(z, bits1, bits2)

    s1 = jnp.concatenate([s1h[:_GA], s1t])[:, :, 0].reshape(_B)
    s2 = jnp.concatenate([s2h[:_GA], s2t])[:, :, 0].reshape(_B)
    return (z, z2, s1, s2)


def kernel(logits):
    return _run(logits)


# speedup vs baseline: 2.0389x; 1.3956x over previous
import functools

import jax
import jax.numpy as jnp
from jax import lax
from jax.experimental import pallas as pl
from jax.experimental.pallas import tpu as pltpu
from jax.experimental.pallas import tpu_sc as plsc

_B = 64
_V = 32768
_K = 8
_BG = 8
_CW = 512
_NC = _V // _CW

_SC_ROWS = 24
_R0 = _B - _SC_ROWS
_GA = _R0 // _BG
_GB = _SC_ROWS // _BG
_QW = _V // 4
_CW1 = 1024
_NC1 = _V // _CW1

_K1 = (1832780943, 270669613)
_K2 = (64467757, 2916123636)

_ROT = (13, 15, 26, 6, 17, 29, 16, 24, 13, 15, 26, 6,
        17, 29, 16, 24, 13, 15, 26, 6)


def _threefry_bits(i, k0, k1):
    ks0 = jnp.uint32(k0)
    ks1 = jnp.uint32(k1)
    ks2 = jnp.uint32(k0 ^ k1 ^ 0x1BD11BDA)
    inject = ((ks1, ks2), (ks2, ks0), (ks0, ks1), (ks1, ks2), (ks2, ks0))
    x0 = jnp.full_like(i, ks0)
    x1 = i + ks1
    for r in range(20):
        rot = _ROT[r]
        x0 = x0 + x1
        x1 = (x1 << jnp.uint32(rot)) | (x1 >> jnp.uint32(32 - rot))
        x1 = x1 ^ x0
        if r % 4 == 3:
            a, b = inject[r // 4]
            x0 = x0 + a
            x1 = x1 + b + jnp.uint32(r // 4 + 1)
    return x0 ^ x1


def _bits_to_gumbel(bits):
    fl = jax.lax.bitcast_convert_type(
        (bits >> jnp.uint32(9)) | jnp.uint32(0x3F800000), jnp.float32)
    fl = fl - jnp.float32(1.0)
    u = jnp.maximum(jnp.float32(1e-20),
                    fl * jnp.float32(1.0) + jnp.float32(1e-20))
    return -jnp.log(-jnp.log(u))


def _sum_mask(x_ref, sl):
    l = None
    for k in range(_K):
        xk = x_ref[:, k, sl]
        xk = jnp.where(jnp.isneginf(xk), 0.0, xk)
        l = xk if l is None else l + xk
    return jnp.where(l == 0.0, -1000.0, l)


def _finalize(m, i):
    gmax = jnp.max(m, axis=1, keepdims=True)
    idx = jnp.min(jnp.where(m == gmax, i, jnp.int32(2**30)),
                  axis=1, keepdims=True)
    return jnp.broadcast_to(idx, (_BG, 128)).astype(jnp.int32)


def _finalize_w(m, i, w):
    return _finalize(m, i)


_NEG = -3.4e38


def _sc_bits_body(o1, o2, buf1, buf2):
    cid = lax.axis_index("c")
    sid = lax.axis_index("s")
    wid = sid * 2 + cid
    iota = lax.iota(jnp.uint32, 16)

    def fill(buf, base, k0, k1):
        def body(j, _):
            b0 = base + (j * jnp.int32(64)).astype(jnp.uint32)
            for t in range(4):
                cvec = b0 + jnp.uint32(t * 16) + iota
                buf[pl.ds(j * 64 + t * 16, 16)] = _threefry_bits(cvec, k0, k1)
            return 0
        lax.fori_loop(0, _QW // 64, body, 0)

    for t in range(3):
        q = wid * 3 + t
        row = q // 4
        off = (q % 4) * _QW
        base = (jnp.uint32(_R0) + row.astype(jnp.uint32)) * jnp.uint32(_V) \
            + off.astype(jnp.uint32)
        fill(buf1, base, *_K1)
        pltpu.sync_copy(buf1, o1.at[row, pl.ds(off, _QW)])
        fill(buf2, base, *_K2)
        pltpu.sync_copy(buf2, o2.at[row, pl.ds(off, _QW)])


_sc_bits = functools.partial(
    pl.kernel,
    out_type=[jax.ShapeDtypeStruct((_SC_ROWS, _V), jnp.uint32),
              jax.ShapeDtypeStruct((_SC_ROWS, _V), jnp.uint32)],
    scratch_types=[pltpu.VMEM((_QW,), jnp.uint32),
                   pltpu.VMEM((_QW,), jnp.uint32)],
    mesh=plsc.VectorSubcoreMesh(core_axis_name="c", subcore_axis_name="s"),
)(_sc_bits_body)


def _tc_a_body(x_ref, z_ref, z2_ref, s1_ref, s2_ref):
    g = pl.program_id(0)
    lin0 = jax.lax.broadcasted_iota(jnp.int32, (_BG, _CW), 1)

    @pl.when(g < _GA)
    def _head():
        row = jax.lax.broadcasted_iota(jnp.uint32, (_BG, _CW1), 0)
        col = jax.lax.broadcasted_iota(jnp.uint32, (_BG, _CW1), 1)
        cnt0 = (jnp.uint32(_BG * _V) * jnp.uint32(g)
                + row * jnp.uint32(_V) + col)
        lin1 = jax.lax.broadcasted_iota(jnp.int32, (_BG, _CW1), 1)

        def loop1(c, carry):
            m1, i1 = carry
            sl = pl.ds(c * _CW1, _CW1)
            z = _sum_mask(x_ref, sl)
            z_ref[:, sl] = z
            z2_ref[:, sl] = z
            b1 = _threefry_bits(cnt0 + jnp.uint32(c * _CW1), *_K1)
            n1 = z + _bits_to_gumbel(b1)
            lin = lin1 + c * _CW1
            up = n1 > m1
            return (jnp.where(up, n1, m1), jnp.where(up, lin, i1))

        def loop2(c, carry):
            m2, i2 = carry
            sl = pl.ds(c * _CW1, _CW1)
            z = z_ref[:, sl]
            b2 = _threefry_bits(cnt0 + jnp.uint32(c * _CW1), *_K2)
            n2 = z + _bits_to_gumbel(b2)
            lin = lin1 + c * _CW1
            up = n2 > m2
            return (jnp.where(up, n2, m2), jnp.where(up, lin, i2))

        init = (jnp.full((_BG, _CW1), _NEG, jnp.float32),
                jnp.zeros((_BG, _CW1), jnp.int32))
        m1, i1 = jax.lax.fori_loop(0, _NC1, loop1, init)
        m2, i2 = jax.lax.fori_loop(0, _NC1, loop2, init)
        s1_ref[0] = _finalize_w(m1, i1, _CW1)
        s2_ref[0] = _finalize_w(m2, i2, _CW1)

    @pl.when(g >= _GA)
    def _tail():
        def loopz(c, carry):
            sl = pl.ds(c * _CW, _CW)
            z = _sum_mask(x_ref, sl)
            z_ref[:, sl] = z
            z2_ref[:, sl] = z
            return carry
        jax.lax.fori_loop(0, _NC, loopz, 0)


def _tc_b_body(z_ref, b1_ref, b2_ref, s1_ref, s2_ref):
    lin0 = jax.lax.broadcasted_iota(jnp.int32, (_BG, _CW), 1)

    def loop(c, carry):
        m1, i1, m2, i2 = carry
        sl = pl.ds(c * _CW, _CW)
        z = z_ref[:, sl]
        lin = lin0 + c * _CW
        n1 = z + _bits_to_gumbel(b1_ref[:, sl])
        up1 = n1 > m1
        m1 = jnp.where(up1, n1, m1)
        i1 = jnp.where(up1, lin, i1)
        n2 = z + _bits_to_gumbel(b2_ref[:, sl])
        up2 = n2 > m2
        m2 = jnp.where(up2, n2, m2)
        i2 = jnp.where(up2, lin, i2)
        return (m1, i1, m2, i2)

    init = (jnp.full((_BG, _CW), _NEG, jnp.float32),
            jnp.zeros((_BG, _CW), jnp.int32),
            jnp.full((_BG, _CW), _NEG, jnp.float32),
            jnp.zeros((_BG, _CW), jnp.int32))
    m1, i1, m2, i2 = jax.lax.fori_loop(0, _NC, loop, init)
    s1_ref[0] = _finalize(m1, i1)
    s2_ref[0] = _finalize(m2, i2)


@functools.partial(jax.jit)
def _run(logits):
    xt = jnp.swapaxes(logits, 1, 2)
    bits1, bits2 = _sc_bits()

    z, z2, s1h, s2h = pl.pallas_call(
        _tc_a_body,
        grid=(_B // _BG,),
        in_specs=[pl.BlockSpec((_BG, _K, _V), lambda g: (g, 0, 0))],
        out_specs=[
            pl.BlockSpec((_BG, _V), lambda g: (g, 0)),
            pl.BlockSpec((_BG, _V), lambda g: (g, 0)),
            pl.BlockSpec((1, _BG, 128), lambda g: (g, 0, 0)),
            pl.BlockSpec((1, _BG, 128), lambda g: (g, 0, 0)),
        ],
        out_shape=[
            jax.ShapeDtypeStruct((_B, _V), jnp.float32),
            jax.ShapeDtypeStruct((_B, _V), jnp.float32),
            jax.ShapeDtypeStruct((_B // _BG, _BG, 128), jnp.int32),
            jax.ShapeDtypeStruct((_B // _BG, _BG, 128), jnp.int32),
        ],
        compiler_params=pltpu.CompilerParams(
            dimension_semantics=("arbitrary",),
        ),
    )(xt)

    s1t, s2t = pl.pallas_call(
        _tc_b_body,
        grid=(_GB,),
        in_specs=[
            pl.BlockSpec((_BG, _V), lambda g: (g + _GA, 0)),
            pl.BlockSpec((_BG, _V), lambda g: (g, 0)),
            pl.BlockSpec((_BG, _V), lambda g: (g, 0)),
        ],
        out_specs=[
            pl.BlockSpec((1, _BG, 128), lambda g: (g, 0, 0)),
            pl.BlockSpec((1, _BG, 128), lambda g: (g, 0, 0)),
        ],
        out_shape=[
            jax.ShapeDtypeStruct((_GB, _BG, 128), jnp.int32),
            jax.ShapeDtypeStruct((_GB, _BG, 128), jnp.int32),
        ],
        compiler_params=pltpu.CompilerParams(
            dimension_semantics=("arbitrary",),
        ),
    )(z, bits1, bits2)

    s1 = jnp.concatenate([s1h[:_GA], s1t])[:, :, 0].reshape(_B)
    s2 = jnp.concatenate([s2h[:_GA], s2t])[:, :, 0].reshape(_B)
    return (z, z2, s1, s2)


def kernel(logits):
    return _run(logits)


# speedup vs baseline: 2.0686x; 1.0145x over previous
import functools

import jax
import jax.numpy as jnp
from jax import lax
from jax.experimental import pallas as pl
from jax.experimental.pallas import tpu as pltpu
from jax.experimental.pallas import tpu_sc as plsc

_B = 64
_V = 32768
_K = 8
_BG = 8
_CW = 512
_NC = _V // _CW

_SC_ROWS = 24
_R0 = _B - _SC_ROWS
_GA = _R0 // _BG
_GB = _SC_ROWS // _BG
_QW = _V // 4
_CW1 = 2048
_NC1 = _V // _CW1

_K1 = (1832780943, 270669613)
_K2 = (64467757, 2916123636)

_ROT = (13, 15, 26, 6, 17, 29, 16, 24, 13, 15, 26, 6,
        17, 29, 16, 24, 13, 15, 26, 6)


def _threefry_bits(i, k0, k1):
    ks0 = jnp.uint32(k0)
    ks1 = jnp.uint32(k1)
    ks2 = jnp.uint32(k0 ^ k1 ^ 0x1BD11BDA)
    inject = ((ks1, ks2), (ks2, ks0), (ks0, ks1), (ks1, ks2), (ks2, ks0))
    x0 = jnp.full_like(i, ks0)
    x1 = i + ks1
    for r in range(20):
        rot = _ROT[r]
        x0 = x0 + x1
        x1 = (x1 << jnp.uint32(rot)) | (x1 >> jnp.uint32(32 - rot))
        x1 = x1 ^ x0
        if r % 4 == 3:
            a, b = inject[r // 4]
            x0 = x0 + a
            x1 = x1 + b + jnp.uint32(r // 4 + 1)
    return x0 ^ x1


def _bits_to_gumbel(bits):
    fl = jax.lax.bitcast_convert_type(
        (bits >> jnp.uint32(9)) | jnp.uint32(0x3F800000), jnp.float32)
    fl = fl - jnp.float32(1.0)
    u = jnp.maximum(jnp.float32(1e-20),
                    fl * jnp.float32(1.0) + jnp.float32(1e-20))
    return -jnp.log(-jnp.log(u))


def _sum_mask(x_ref, sl):
    l = None
    for k in range(_K):
        xk = x_ref[:, k, sl]
        xk = jnp.where(jnp.isneginf(xk), 0.0, xk)
        l = xk if l is None else l + xk
    return jnp.where(l == 0.0, -1000.0, l)


def _finalize(m, i):
    gmax = jnp.max(m, axis=1, keepdims=True)
    idx = jnp.min(jnp.where(m == gmax, i, jnp.int32(2**30)),
                  axis=1, keepdims=True)
    return jnp.broadcast_to(idx, (_BG, 128)).astype(jnp.int32)


def _finalize_w(m, i, w):
    return _finalize(m, i)


_NEG = -3.4e38


def _sc_bits_body(o1, o2, buf1, buf2):
    cid = lax.axis_index("c")
    sid = lax.axis_index("s")
    wid = sid * 2 + cid
    iota = lax.iota(jnp.uint32, 16)

    def fill(buf, base, k0, k1):
        def body(j, _):
            b0 = base + (j * jnp.int32(64)).astype(jnp.uint32)
            for t in range(4):
                cvec = b0 + jnp.uint32(t * 16) + iota
                buf[pl.ds(j * 64 + t * 16, 16)] = _threefry_bits(cvec, k0, k1)
            return 0
        lax.fori_loop(0, _QW // 64, body, 0)

    for t in range(3):
        q = wid * 3 + t
        row = q // 4
        off = (q % 4) * _QW
        base = (jnp.uint32(_R0) + row.astype(jnp.uint32)) * jnp.uint32(_V) \
            + off.astype(jnp.uint32)
        fill(buf1, base, *_K1)
        pltpu.sync_copy(buf1, o1.at[row, pl.ds(off, _QW)])
        fill(buf2, base, *_K2)
        pltpu.sync_copy(buf2, o2.at[row, pl.ds(off, _QW)])


_sc_bits = functools.partial(
    pl.kernel,
    out_type=[jax.ShapeDtypeStruct((_SC_ROWS, _V), jnp.uint32),
              jax.ShapeDtypeStruct((_SC_ROWS, _V), jnp.uint32)],
    scratch_types=[pltpu.VMEM((_QW,), jnp.uint32),
                   pltpu.VMEM((_QW,), jnp.uint32)],
    mesh=plsc.VectorSubcoreMesh(core_axis_name="c", subcore_axis_name="s"),
)(_sc_bits_body)


def _tc_a_body(x_ref, z_ref, z2_ref, s1_ref, s2_ref):
    g = pl.program_id(0)
    lin0 = jax.lax.broadcasted_iota(jnp.int32, (_BG, _CW), 1)

    @pl.when(g < _GA)
    def _head():
        row = jax.lax.broadcasted_iota(jnp.uint32, (_BG, _CW1), 0)
        col = jax.lax.broadcasted_iota(jnp.uint32, (_BG, _CW1), 1)
        cnt0 = (jnp.uint32(_BG * _V) * jnp.uint32(g)
                + row * jnp.uint32(_V) + col)
        lin1 = jax.lax.broadcasted_iota(jnp.int32, (_BG, _CW1), 1)

        def loop1(c, carry):
            m1, i1 = carry
            sl = pl.ds(c * _CW1, _CW1)
            z = _sum_mask(x_ref, sl)
            z_ref[:, sl] = z
            z2_ref[:, sl] = z
            b1 = _threefry_bits(cnt0 + jnp.uint32(c * _CW1), *_K1)
            n1 = z + _bits_to_gumbel(b1)
            lin = lin1 + c * _CW1
            up = n1 > m1
            return (jnp.where(up, n1, m1), jnp.where(up, lin, i1))

        def loop2(c, carry):
            m2, i2 = carry
            sl = pl.ds(c * _CW1, _CW1)
            z = z_ref[:, sl]
            b2 = _threefry_bits(cnt0 + jnp.uint32(c * _CW1), *_K2)
            n2 = z + _bits_to_gumbel(b2)
            lin = lin1 + c * _CW1
            up = n2 > m2
            return (jnp.where(up, n2, m2), jnp.where(up, lin, i2))

        init = (jnp.full((_BG, _CW1), _NEG, jnp.float32),
                jnp.zeros((_BG, _CW1), jnp.int32))
        m1, i1 = jax.lax.fori_loop(0, _NC1, loop1, init)
        m2, i2 = jax.lax.fori_loop(0, _NC1, loop2, init)
        s1_ref[0] = _finalize_w(m1, i1, _CW1)
        s2_ref[0] = _finalize_w(m2, i2, _CW1)

    @pl.when(g >= _GA)
    def _tail():
        def loopz(c, carry):
            sl = pl.ds(c * _CW, _CW)
            z = _sum_mask(x_ref, sl)
            z_ref[:, sl] = z
            z2_ref[:, sl] = z
            return carry
        jax.lax.fori_loop(0, _NC, loopz, 0)


def _tc_b_body(z_ref, b1_ref, b2_ref, s1_ref, s2_ref):
    lin0 = jax.lax.broadcasted_iota(jnp.int32, (_BG, _CW), 1)

    def loop(c, carry):
        m1, i1, m2, i2 = carry
        sl = pl.ds(c * _CW, _CW)
        z = z_ref[:, sl]
        lin = lin0 + c * _CW
        n1 = z + _bits_to_gumbel(b1_ref[:, sl])
        up1 = n1 > m1
        m1 = jnp.where(up1, n1, m1)
        i1 = jnp.where(up1, lin, i1)
        n2 = z + _bits_to_gumbel(b2_ref[:, sl])
        up2 = n2 > m2
        m2 = jnp.where(up2, n2, m2)
        i2 = jnp.where(up2, lin, i2)
        return (m1, i1, m2, i2)

    init = (jnp.full((_BG, _CW), _NEG, jnp.float32),
            jnp.zeros((_BG, _CW), jnp.int32),
            jnp.full((_BG, _CW), _NEG, jnp.float32),
            jnp.zeros((_BG, _CW), jnp.int32))
    m1, i1, m2, i2 = jax.lax.fori_loop(0, _NC, loop, init)
    s1_ref[0] = _finalize(m1, i1)
    s2_ref[0] = _finalize(m2, i2)


@functools.partial(jax.jit)
def _run(logits):
    xt = jnp.swapaxes(logits, 1, 2)
    bits1, bits2 = _sc_bits()

    z, z2, s1h, s2h = pl.pallas_call(
        _tc_a_body,
        grid=(_B // _BG,),
        in_specs=[pl.BlockSpec((_BG, _K, _V), lambda g: (g, 0, 0))],
        out_specs=[
            pl.BlockSpec((_BG, _V), lambda g: (g, 0)),
            pl.BlockSpec((_BG, _V), lambda g: (g, 0)),
            pl.BlockSpec((1, _BG, 128), lambda g: (g, 0, 0)),
            pl.BlockSpec((1, _BG, 128), lambda g: (g, 0, 0)),
        ],
        out_shape=[
            jax.ShapeDtypeStruct((_B, _V), jnp.float32),
            jax.ShapeDtypeStruct((_B, _V), jnp.float32),
            jax.ShapeDtypeStruct((_B // _BG, _BG, 128), jnp.int32),
            jax.ShapeDtypeStruct((_B // _BG, _BG, 128), jnp.int32),
        ],
        compiler_params=pltpu.CompilerParams(
            dimension_semantics=("arbitrary",),
        ),
    )(xt)

    s1t, s2t = pl.pallas_call(
        _tc_b_body,
        grid=(_GB,),
        in_specs=[
            pl.BlockSpec((_BG, _V), lambda g: (g + _GA, 0)),
            pl.BlockSpec((_BG, _V), lambda g: (g, 0)),
            pl.BlockSpec((_BG, _V), lambda g: (g, 0)),
        ],
        out_specs=[
            pl.BlockSpec((1, _BG, 128), lambda g: (g, 0, 0)),
            pl.BlockSpec((1, _BG, 128), lambda g: (g, 0, 0)),
        ],
        out_shape=[
            jax.ShapeDtypeStruct((_GB, _BG, 128), jnp.int32),
            jax.ShapeDtypeStruct((_GB, _BG, 128), jnp.int32),
        ],
        compiler_params=pltpu.CompilerParams(
            dimension_semantics=("arbitrary",),
        ),
    )(z, bits1, bits2)

    s1 = jnp.concatenate([s1h[:_GA], s1t])[:, :, 0].reshape(_B)
    s2 = jnp.concatenate([s2h[:_GA], s2t])[:, :, 0].reshape(_B)
    return (z, z2, s1, s2)


def kernel(logits):
    return _run(logits)
